# baseline (speedup 1.0000x reference)
import functools

import jax
import jax.numpy as jnp
from jax import lax
from jax.experimental import pallas as pl
from jax.experimental.pallas import tpu as pltpu
from jax.experimental.pallas import tpu_sc as plsc

VOCAB_SIZE = 100000
EMB_DIM = 128
BATCH_SIZE = 16384


def _make_sc_kernel(batch, dim):
    info = plsc.get_sparse_core_info()
    nc, ns, lanes = info.num_cores, info.num_subcores, info.num_lanes
    nw = nc * ns
    b_per_w = batch // nw
    chunk = 128
    n_chunks = b_per_w // chunk
    groups = chunk // lanes
    nbuf = 3

    mesh = plsc.VectorSubcoreMesh(core_axis_name="c", subcore_axis_name="s")

    @functools.partial(
        pl.kernel,
        mesh=mesh,
        compiler_params=pltpu.CompilerParams(needs_layout_passes=False),
        out_type=jax.ShapeDtypeStruct((batch,), jnp.float32),
        scratch_types=[
            pltpu.VMEM((b_per_w,), jnp.int32),
            pltpu.VMEM((b_per_w,), jnp.int32),
            pltpu.VMEM((nbuf, chunk, dim), jnp.float32),
            pltpu.VMEM((nbuf, chunk, dim), jnp.float32),
            pltpu.VMEM((b_per_w,), jnp.float32),
        ] + [pltpu.SemaphoreType.DMA] * (nbuf + 1),
    )
    def kern(iw_hbm, tw_hbm, ie_hbm, oe_hbm, out_hbm,
             idx_i, idx_t, rows_i, rows_t, scores_v, *sems):
        wid = lax.axis_index("s") * nc + lax.axis_index("c")
        base = wid * b_per_w
        row_iota = lax.iota(jnp.int32, lanes)

        out_sem = sems[nbuf]
        cp_ii = pltpu.async_copy(iw_hbm.at[pl.ds(base, b_per_w)], idx_i,
                                 out_sem)
        cp_tt = pltpu.async_copy(tw_hbm.at[pl.ds(base, b_per_w)], idx_t,
                                 out_sem)
        cp_ii.wait()
        cp_tt.wait()

        def issue(ci):
            slot = ci % nbuf
            sem = sems[slot]
            cp_i = pltpu.async_copy(
                ie_hbm.at[idx_i.at[pl.ds(ci * chunk, chunk)]],
                rows_i.at[slot], sem)
            cp_t = pltpu.async_copy(
                oe_hbm.at[idx_t.at[pl.ds(ci * chunk, chunk)]],
                rows_t.at[slot], sem)
            return (cp_i, cp_t)

        cps = {}
        out_cps = []
        for ci in range(min(nbuf - 1, n_chunks)):
            cps[ci] = issue(ci)
        for ci in range(n_chunks):
            if ci + nbuf - 1 < n_chunks:
                cps[ci + nbuf - 1] = issue(ci + nbuf - 1)
            for cp in cps.pop(ci):
                cp.wait()
            slot = ci % nbuf
            ri = rows_i.at[slot]
            rt = rows_t.at[slot]

            nskew = dim // lanes
            pats = [(((row_iota >> 1) + c) & (nskew - 1)) * lanes
                    for c in range(nskew)]
            rows_g = [row_iota + g * lanes for g in range(groups)]

            def w_body(w, accs, ri=ri, rt=rt):
                accs = list(accs)
                for c in range(nskew):
                    cols = pats[c] + w
                    for g in range(groups):
                        a = plsc.load_gather(ri, [rows_g[g], cols])
                        b = plsc.load_gather(rt, [rows_g[g], cols])
                        accs[g] = accs[g] + a * b
                return tuple(accs)

            accs = lax.fori_loop(
                0, lanes, w_body,
                tuple(jnp.zeros((lanes,), jnp.float32)
                      for _ in range(groups)))
            for g in range(groups):
                scores_v[pl.ds(ci * chunk + g * lanes, lanes)] = accs[g]
            out_cps.append(pltpu.async_copy(
                scores_v.at[pl.ds(ci * chunk, chunk)],
                out_hbm.at[pl.ds(base + ci * chunk, chunk)], out_sem))

        for cp in out_cps:
            cp.wait()

    return kern


def kernel(input_words, target_words, in_embed, out_embed):
    batch = input_words.shape[0]
    dim = in_embed.shape[1]
    kern = _make_sc_kernel(batch, dim)
    return kern(input_words.astype(jnp.int32), target_words.astype(jnp.int32),
                in_embed, out_embed)


# speedup vs baseline: 2.0573x; 2.0573x over previous
import functools

import jax
import jax.numpy as jnp
from jax import lax
from jax.experimental import pallas as pl
from jax.experimental.pallas import tpu as pltpu
from jax.experimental.pallas import tpu_sc as plsc

VOCAB_SIZE = 100000
EMB_DIM = 128
BATCH_SIZE = 16384


def _make_sc_kernel(batch, dim):
    info = plsc.get_sparse_core_info()
    nc, ns, lanes = info.num_cores, info.num_subcores, info.num_lanes
    nw = nc * ns
    b_per_w = batch // nw
    chunk = 128
    n_chunks = b_per_w // chunk
    groups = chunk // lanes
    nbuf = 3

    mesh = plsc.VectorSubcoreMesh(core_axis_name="c", subcore_axis_name="s")

    @functools.partial(
        pl.kernel,
        mesh=mesh,
        compiler_params=pltpu.CompilerParams(needs_layout_passes=False),
        out_type=jax.ShapeDtypeStruct((batch,), jnp.float32),
        scratch_types=[
            pltpu.VMEM((b_per_w,), jnp.int32),
            pltpu.VMEM((b_per_w,), jnp.int32),
            pltpu.VMEM((nbuf, chunk, dim), jnp.float32),
            pltpu.VMEM((nbuf, chunk, dim), jnp.float32),
            pltpu.VMEM((b_per_w,), jnp.float32),
        ] + [pltpu.SemaphoreType.DMA] * (nbuf + 1),
    )
    def kern(iw_hbm, tw_hbm, ie_hbm, oe_hbm, out_hbm,
             idx_i, idx_t, rows_i, rows_t, scores_v, *sems):
        wid = lax.axis_index("s") * nc + lax.axis_index("c")
        base = wid * b_per_w
        row_iota = lax.iota(jnp.int32, lanes)

        out_sem = sems[nbuf]
        cp_ii = pltpu.async_copy(iw_hbm.at[pl.ds(base, b_per_w)], idx_i,
                                 out_sem)
        cp_tt = pltpu.async_copy(tw_hbm.at[pl.ds(base, b_per_w)], idx_t,
                                 out_sem)
        cp_ii.wait()
        cp_tt.wait()

        def issue(ci):
            slot = ci % nbuf
            sem = sems[slot]

            def r_body(g16, c):
                iv = idx_i[pl.ds(ci * chunk + g16 * lanes, lanes)]
                tv = idx_t[pl.ds(ci * chunk + g16 * lanes, lanes)]
                for r2 in range(lanes):
                    r = g16 * lanes + r2
                    pltpu.async_copy(ie_hbm.at[pl.ds(iv[r2], 1)],
                                     rows_i.at[slot].at[pl.ds(r, 1)], sem)
                    pltpu.async_copy(oe_hbm.at[pl.ds(tv[r2], 1)],
                                     rows_t.at[slot].at[pl.ds(r, 1)], sem)
                return c

            lax.fori_loop(0, chunk // lanes, r_body, 0)

        def wait_chunk(ci):
            slot = ci % nbuf
            sem = sems[slot]

            def w_body(r, c):
                pltpu.make_async_copy(
                    ie_hbm.at[pl.ds(0, 1)],
                    rows_i.at[slot].at[pl.ds(0, 1)], sem).wait()
                return c

            lax.fori_loop(0, 2 * chunk, w_body, 0)

        out_cps = []
        for ci in range(min(nbuf - 1, n_chunks)):
            issue(ci)
        for ci in range(n_chunks):
            if ci + nbuf - 1 < n_chunks:
                issue(ci + nbuf - 1)
            wait_chunk(ci)
            slot = ci % nbuf
            ri = rows_i.at[slot]
            rt = rows_t.at[slot]

            def group_body(g, gcarry, ri=ri, rt=rt):
                def row_body(r2, vec):
                    r = g * lanes + r2
                    prods = []
                    for k in range(dim // lanes):
                        iv = ri[r, pl.ds(k * lanes, lanes)]
                        ov = rt[r, pl.ds(k * lanes, lanes)]
                        prods.append(iv * ov)
                    while len(prods) > 1:
                        prods = [a + b for a, b in
                                 zip(prods[::2], prods[1::2])]
                    s = jnp.sum(prods[0])
                    return jnp.where(row_iota == r2, s, vec)

                vec = lax.fori_loop(0, lanes, row_body,
                                    jnp.zeros((lanes,), jnp.float32),
                                    unroll=4)
                scores_v[pl.ds(ci * chunk + g * lanes, lanes)] = vec
                return gcarry

            lax.fori_loop(0, groups, group_body, 0)
            out_cps.append(pltpu.async_copy(
                scores_v.at[pl.ds(ci * chunk, chunk)],
                out_hbm.at[pl.ds(base + ci * chunk, chunk)], out_sem))

        for cp in out_cps:
            cp.wait()

    return kern


def kernel(input_words, target_words, in_embed, out_embed):
    batch = input_words.shape[0]
    dim = in_embed.shape[1]
    kern = _make_sc_kernel(batch, dim)
    return kern(input_words.astype(jnp.int32), target_words.astype(jnp.int32),
                in_embed, out_embed)


# speedup vs baseline: 2.1851x; 1.0621x over previous
import functools

import jax
import jax.numpy as jnp
from jax import lax
from jax.experimental import pallas as pl
from jax.experimental.pallas import tpu as pltpu
from jax.experimental.pallas import tpu_sc as plsc

VOCAB_SIZE = 100000
EMB_DIM = 128
BATCH_SIZE = 16384


def _make_sc_kernel(batch, dim):
    info = plsc.get_sparse_core_info()
    nc, ns, lanes = info.num_cores, info.num_subcores, info.num_lanes
    nw = nc * ns
    b_per_w = batch // nw
    chunk = 128
    n_chunks = b_per_w // chunk
    groups = chunk // lanes
    nbuf = 3
    dma_rows = chunk // 2

    mesh = plsc.VectorSubcoreMesh(core_axis_name="c", subcore_axis_name="s")

    @functools.partial(
        pl.kernel,
        mesh=mesh,
        compiler_params=pltpu.CompilerParams(needs_layout_passes=False),
        out_type=jax.ShapeDtypeStruct((batch,), jnp.float32),
        scratch_types=[
            pltpu.VMEM((b_per_w,), jnp.int32),
            pltpu.VMEM((b_per_w,), jnp.int32),
            pltpu.VMEM((nbuf, chunk, dim), jnp.float32),
            pltpu.VMEM((nbuf, chunk, dim), jnp.float32),
            pltpu.VMEM((b_per_w,), jnp.float32),
        ] + [pltpu.SemaphoreType.DMA] * (2 * nbuf + 1),
    )
    def kern(iw_hbm, tw_hbm, ie_hbm, oe_hbm, out_hbm,
             idx_i, idx_t, rows_i, rows_t, scores_v, *sems):
        wid = lax.axis_index("s") * nc + lax.axis_index("c")
        base = wid * b_per_w
        row_iota = lax.iota(jnp.int32, lanes)

        out_sem = sems[2 * nbuf]
        cp_ii = pltpu.async_copy(iw_hbm.at[pl.ds(base, b_per_w)], idx_i,
                                 out_sem)
        cp_tt = pltpu.async_copy(tw_hbm.at[pl.ds(base, b_per_w)], idx_t,
                                 out_sem)
        cp_ii.wait()
        cp_tt.wait()

        def issue(ci):
            slot = ci % nbuf
            dsem = sems[nbuf + slot]

            def r_body(g16, c):
                iv = idx_i[pl.ds(ci * chunk + g16 * lanes, lanes)]
                tv = idx_t[pl.ds(ci * chunk + g16 * lanes, lanes)]
                for r2 in range(lanes):
                    r = g16 * lanes + r2
                    pltpu.async_copy(ie_hbm.at[pl.ds(iv[r2], 1)],
                                     rows_i.at[slot].at[pl.ds(r, 1)], dsem)
                    pltpu.async_copy(oe_hbm.at[pl.ds(tv[r2], 1)],
                                     rows_t.at[slot].at[pl.ds(r, 1)], dsem)
                return c

            lax.fori_loop(0, dma_rows // lanes, r_body, 0)

            ssem = sems[slot]
            srows = chunk - dma_rows
            cp_i = pltpu.async_copy(
                ie_hbm.at[idx_i.at[pl.ds(ci * chunk + dma_rows, srows)]],
                rows_i.at[slot].at[pl.ds(dma_rows, srows)], ssem)
            cp_t = pltpu.async_copy(
                oe_hbm.at[idx_t.at[pl.ds(ci * chunk + dma_rows, srows)]],
                rows_t.at[slot].at[pl.ds(dma_rows, srows)], ssem)
            return (cp_i, cp_t)

        def wait_chunk(ci, cps):
            slot = ci % nbuf
            for cp in cps:
                cp.wait()
            dsem = sems[nbuf + slot]

            def w_body(r, c):
                pltpu.make_async_copy(
                    ie_hbm.at[pl.ds(0, 1)],
                    rows_i.at[slot].at[pl.ds(0, 1)], dsem).wait()
                return c

            lax.fori_loop(0, 2 * dma_rows, w_body, 0)

        cps = {}
        out_cps = []
        for ci in range(min(nbuf - 1, n_chunks)):
            cps[ci] = issue(ci)
        for ci in range(n_chunks):
            if ci + nbuf - 1 < n_chunks:
                cps[ci + nbuf - 1] = issue(ci + nbuf - 1)
            wait_chunk(ci, cps.pop(ci))
            slot = ci % nbuf
            ri = rows_i.at[slot]
            rt = rows_t.at[slot]

            def group_body(g, gcarry, ri=ri, rt=rt):
                def row_body(r2, vec):
                    r = g * lanes + r2
                    prods = []
                    for k in range(dim // lanes):
                        iv = ri[r, pl.ds(k * lanes, lanes)]
                        ov = rt[r, pl.ds(k * lanes, lanes)]
                        prods.append(iv * ov)
                    while len(prods) > 1:
                        prods = [a + b for a, b in
                                 zip(prods[::2], prods[1::2])]
                    s = jnp.sum(prods[0])
                    return jnp.where(row_iota == r2, s, vec)

                vec = lax.fori_loop(0, lanes, row_body,
                                    jnp.zeros((lanes,), jnp.float32),
                                    unroll=4)
                scores_v[pl.ds(ci * chunk + g * lanes, lanes)] = vec
                return gcarry

            lax.fori_loop(0, groups, group_body, 0)
            out_cps.append(pltpu.async_copy(
                scores_v.at[pl.ds(ci * chunk, chunk)],
                out_hbm.at[pl.ds(base + ci * chunk, chunk)], out_sem))

        for cp in out_cps:
            cp.wait()

    return kern


def kernel(input_words, target_words, in_embed, out_embed):
    batch = input_words.shape[0]
    dim = in_embed.shape[1]
    kern = _make_sc_kernel(batch, dim)
    return kern(input_words.astype(jnp.int32), target_words.astype(jnp.int32),
                in_embed, out_embed)


# speedup vs baseline: 2.4217x; 1.1082x over previous
import functools

import jax
import jax.numpy as jnp
from jax import lax
from jax.experimental import pallas as pl
from jax.experimental.pallas import tpu as pltpu
from jax.experimental.pallas import tpu_sc as plsc

VOCAB_SIZE = 100000
EMB_DIM = 128
BATCH_SIZE = 16384


def _make_sc_kernel(batch, dim):
    info = plsc.get_sparse_core_info()
    nc, ns, lanes = info.num_cores, info.num_subcores, info.num_lanes
    nw = nc * ns
    b_per_w = batch // nw
    chunk = 128
    n_chunks = b_per_w // chunk
    groups = chunk // lanes
    nbuf = 3

    mesh = plsc.VectorSubcoreMesh(core_axis_name="c", subcore_axis_name="s")

    @functools.partial(
        pl.kernel,
        mesh=mesh,
        compiler_params=pltpu.CompilerParams(needs_layout_passes=False),
        out_type=jax.ShapeDtypeStruct((batch,), jnp.float32),
        scratch_types=[
            pltpu.VMEM((b_per_w,), jnp.int32),
            pltpu.VMEM((b_per_w,), jnp.int32),
            pltpu.VMEM((nbuf, chunk, dim), jnp.float32),
            pltpu.VMEM((nbuf, chunk, dim), jnp.float32),
            pltpu.VMEM((b_per_w,), jnp.float32),
        ] + [pltpu.SemaphoreType.DMA] * (nbuf + 1),
    )
    def kern(iw_hbm, tw_hbm, ie_hbm, oe_hbm, out_hbm,
             idx_i, idx_t, rows_i, rows_t, scores_v, *sems):
        wid = lax.axis_index("s") * nc + lax.axis_index("c")
        base = wid * b_per_w
        row_iota = lax.iota(jnp.int32, lanes)

        out_sem = sems[nbuf]
        cp_ii = pltpu.async_copy(iw_hbm.at[pl.ds(base, b_per_w)], idx_i,
                                 out_sem)
        cp_tt = pltpu.async_copy(tw_hbm.at[pl.ds(base, b_per_w)], idx_t,
                                 out_sem)
        cp_ii.wait()
        cp_tt.wait()

        def issue(ci):
            slot = ci % nbuf
            sem = sems[slot]
            cp_i = pltpu.async_copy(
                ie_hbm.at[idx_i.at[pl.ds(ci * chunk, chunk)]],
                rows_i.at[slot], sem)
            cp_t = pltpu.async_copy(
                oe_hbm.at[idx_t.at[pl.ds(ci * chunk, chunk)]],
                rows_t.at[slot], sem)
            return (cp_i, cp_t)

        cps = {}
        out_cps = []
        for ci in range(min(nbuf - 1, n_chunks)):
            cps[ci] = issue(ci)
        for ci in range(n_chunks):
            if ci + nbuf - 1 < n_chunks:
                cps[ci + nbuf - 1] = issue(ci + nbuf - 1)
            for cp in cps.pop(ci):
                cp.wait()
            slot = ci % nbuf
            ri = rows_i.at[slot]
            rt = rows_t.at[slot]

            @plsc.parallel_loop(0, groups)
            def group_body(g, ri=ri, rt=rt):
                def row_body(r2, vec):
                    r = g * lanes + r2
                    prods = []
                    for k in range(dim // lanes):
                        iv = ri[r, pl.ds(k * lanes, lanes)]
                        ov = rt[r, pl.ds(k * lanes, lanes)]
                        prods.append(iv * ov)
                    while len(prods) > 1:
                        prods = [a + b for a, b in
                                 zip(prods[::2], prods[1::2])]
                    s = jnp.sum(prods[0])
                    return jnp.where(row_iota == r2, s, vec)

                vec = lax.fori_loop(0, lanes, row_body,
                                    jnp.zeros((lanes,), jnp.float32),
                                    unroll=4)
                scores_v[pl.ds(ci * chunk + g * lanes, lanes)] = vec
            out_cps.append(pltpu.async_copy(
                scores_v.at[pl.ds(ci * chunk, chunk)],
                out_hbm.at[pl.ds(base + ci * chunk, chunk)], out_sem))

        for cp in out_cps:
            cp.wait()

    return kern


def kernel(input_words, target_words, in_embed, out_embed):
    batch = input_words.shape[0]
    dim = in_embed.shape[1]
    kern = _make_sc_kernel(batch, dim)
    return kern(input_words.astype(jnp.int32), target_words.astype(jnp.int32),
                in_embed, out_embed)
